# baseline (speedup 1.0000x reference)
import jax
import jax.numpy as jnp
from jax.experimental import pallas as pl
from jax.experimental.pallas import tpu as pltpu

_N = 8192
_TA = 1024
_NI = _N // _TA


def _loss_kernel(a_ref, bT_ref, p3T_ref, out_ref, colmin_ref):
    c = pl.program_id(0)
    i = pl.program_id(1)

    a = a_ref[0]
    bT = bT_ref[...]

    ab = (-2.0 * a).astype(jnp.bfloat16)
    bb = bT.astype(jnp.bfloat16)
    a2 = jnp.sum(a * a, axis=1, keepdims=True)
    b2 = jnp.sum(bT * bT, axis=0, keepdims=True)

    w = jnp.where(c == 0, 0.5, 1.0)

    @pl.when(jnp.logical_and(c == 0, i == 0))
    def _init_out():
        diff = p3T_ref[...] - bT
        out_ref[0, 0] = jnp.sum(diff * diff) / (_N * 3)

    @pl.when(i == 0)
    def _init_colmin():
        colmin_ref[...] = jnp.full((1, _N), jnp.inf, jnp.float32)

    row_min = jnp.full((_TA, 1), jnp.inf, jnp.float32)
    for lo in range(0, _N, _N // 8):
        hi = lo + _N // 8
        m = jax.lax.dot_general(
            ab, bb[:, lo:hi],
            dimension_numbers=(((1,), (0,)), ((), ())),
            preferred_element_type=jnp.float32,
        )
        t = m + b2[:, lo:hi]
        row_min = jnp.minimum(row_min, jnp.min(t, axis=1, keepdims=True))
        u = t + a2
        colmin_ref[:, lo:hi] = jnp.minimum(
            colmin_ref[:, lo:hi], jnp.min(u, axis=0, keepdims=True)
        )
    row_min = jnp.maximum(row_min + a2, 0.0)

    total = w * jnp.sum(jnp.sqrt(row_min)) / _N
    out_ref[0, 0] = out_ref[0, 0] + total

    @pl.when(i == _NI - 1)
    def _finish_col():
        col_final = jnp.maximum(colmin_ref[...], 0.0)
        col_sum = jnp.sum(jnp.sqrt(col_final)) / _N
        out_ref[0, 0] = out_ref[0, 0] + w * col_sum


def kernel(pc1, pc2):
    a01 = pc1[:2]
    bT = pc2.T
    p3T = pc1[3].T

    out = pl.pallas_call(
        _loss_kernel,
        grid=(2, _NI),
        in_specs=[
            pl.BlockSpec((1, _TA, 3), lambda c, i: (c, i, 0)),
            pl.BlockSpec((3, _N), lambda c, i: (0, 0)),
            pl.BlockSpec((3, _N), lambda c, i: (0, 0)),
        ],
        out_specs=pl.BlockSpec(memory_space=pltpu.SMEM),
        out_shape=jax.ShapeDtypeStruct((1, 1), jnp.float32),
        scratch_shapes=[pltpu.VMEM((1, _N), jnp.float32)],
        compiler_params=pltpu.CompilerParams(
            dimension_semantics=("arbitrary", "arbitrary"),
        ),
    )(a01, bT, p3T)
    return out[0, 0]


# speedup vs baseline: 1.0592x; 1.0592x over previous
import jax
import jax.numpy as jnp
from jax.experimental import pallas as pl
from jax.experimental.pallas import tpu as pltpu

_N = 8192
_TA = 2048
_NI = _N // _TA


def _loss_kernel(a_ref, bT_ref, p3T_ref, out_ref, colmin_ref):
    c = pl.program_id(0)
    i = pl.program_id(1)

    a = a_ref[0]
    bT = bT_ref[...]

    ab = (-2.0 * a).astype(jnp.bfloat16)
    bb = bT.astype(jnp.bfloat16)
    a2 = jnp.sum(a * a, axis=1, keepdims=True)
    b2 = jnp.sum(bT * bT, axis=0, keepdims=True)

    w = jnp.where(c == 0, 0.5, 1.0)

    @pl.when(jnp.logical_and(c == 0, i == 0))
    def _init_out():
        diff = p3T_ref[...] - bT
        out_ref[0, 0] = jnp.sum(diff * diff) / (_N * 3)

    @pl.when(i == 0)
    def _init_colmin():
        colmin_ref[...] = jnp.full((1, _N), jnp.inf, jnp.float32)

    row_min = jnp.full((_TA, 1), jnp.inf, jnp.float32)
    for lo in range(0, _N, _N // 4):
        hi = lo + _N // 4
        m = jax.lax.dot_general(
            ab, bb[:, lo:hi],
            dimension_numbers=(((1,), (0,)), ((), ())),
            preferred_element_type=jnp.float32,
        )
        t = m + b2[:, lo:hi]
        row_min = jnp.minimum(row_min, jnp.min(t, axis=1, keepdims=True))
        u = t + a2
        colmin_ref[:, lo:hi] = jnp.minimum(
            colmin_ref[:, lo:hi], jnp.min(u, axis=0, keepdims=True)
        )
    row_min = jnp.maximum(row_min + a2, 0.0)

    total = w * jnp.sum(jnp.sqrt(row_min)) / _N
    out_ref[0, 0] = out_ref[0, 0] + total

    @pl.when(i == _NI - 1)
    def _finish_col():
        col_final = jnp.maximum(colmin_ref[...], 0.0)
        col_sum = jnp.sum(jnp.sqrt(col_final)) / _N
        out_ref[0, 0] = out_ref[0, 0] + w * col_sum


def kernel(pc1, pc2):
    a01 = pc1[:2]
    bT = pc2.T
    p3T = pc1[3].T

    out = pl.pallas_call(
        _loss_kernel,
        grid=(2, _NI),
        in_specs=[
            pl.BlockSpec((1, _TA, 3), lambda c, i: (c, i, 0)),
            pl.BlockSpec((3, _N), lambda c, i: (0, 0)),
            pl.BlockSpec((3, _N), lambda c, i: (0, 0)),
        ],
        out_specs=pl.BlockSpec(memory_space=pltpu.SMEM),
        out_shape=jax.ShapeDtypeStruct((1, 1), jnp.float32),
        scratch_shapes=[pltpu.VMEM((1, _N), jnp.float32)],
        compiler_params=pltpu.CompilerParams(
            dimension_semantics=("arbitrary", "arbitrary"),
        ),
    )(a01, bT, p3T)
    return out[0, 0]


# speedup vs baseline: 1.0646x; 1.0051x over previous
import jax
import jax.numpy as jnp
from jax.experimental import pallas as pl
from jax.experimental.pallas import tpu as pltpu

_N = 8192
_TA = 4096
_NI = _N // _TA


def _loss_kernel(a_ref, bT_ref, p3T_ref, out_ref, colmin_ref):
    c = pl.program_id(0)
    i = pl.program_id(1)

    a = a_ref[0]
    bT = bT_ref[...]

    ab = (-2.0 * a).astype(jnp.bfloat16)
    bb = bT.astype(jnp.bfloat16)
    a2 = jnp.sum(a * a, axis=1, keepdims=True)
    b2 = jnp.sum(bT * bT, axis=0, keepdims=True)

    w = jnp.where(c == 0, 0.5, 1.0)

    @pl.when(jnp.logical_and(c == 0, i == 0))
    def _init_out():
        diff = p3T_ref[...] - bT
        out_ref[0, 0] = jnp.sum(diff * diff) / (_N * 3)

    @pl.when(i == 0)
    def _init_colmin():
        colmin_ref[...] = jnp.full((1, _N), jnp.inf, jnp.float32)

    row_min = jnp.full((_TA, 1), jnp.inf, jnp.float32)
    for lo in range(0, _N, _N // 4):
        hi = lo + _N // 4
        m = jax.lax.dot_general(
            ab, bb[:, lo:hi],
            dimension_numbers=(((1,), (0,)), ((), ())),
            preferred_element_type=jnp.float32,
        )
        t = m + b2[:, lo:hi]
        row_min = jnp.minimum(row_min, jnp.min(t, axis=1, keepdims=True))
        u = t + a2
        colmin_ref[:, lo:hi] = jnp.minimum(
            colmin_ref[:, lo:hi], jnp.min(u, axis=0, keepdims=True)
        )
    row_min = jnp.maximum(row_min + a2, 0.0)

    total = w * jnp.sum(jnp.sqrt(row_min)) / _N
    out_ref[0, 0] = out_ref[0, 0] + total

    @pl.when(i == _NI - 1)
    def _finish_col():
        col_final = jnp.maximum(colmin_ref[...], 0.0)
        col_sum = jnp.sum(jnp.sqrt(col_final)) / _N
        out_ref[0, 0] = out_ref[0, 0] + w * col_sum


def kernel(pc1, pc2):
    a01 = pc1[:2]
    bT = pc2.T
    p3T = pc1[3].T

    out = pl.pallas_call(
        _loss_kernel,
        grid=(2, _NI),
        in_specs=[
            pl.BlockSpec((1, _TA, 3), lambda c, i: (c, i, 0)),
            pl.BlockSpec((3, _N), lambda c, i: (0, 0)),
            pl.BlockSpec((3, _N), lambda c, i: (0, 0)),
        ],
        out_specs=pl.BlockSpec(memory_space=pltpu.SMEM),
        out_shape=jax.ShapeDtypeStruct((1, 1), jnp.float32),
        scratch_shapes=[pltpu.VMEM((1, _N), jnp.float32)],
        compiler_params=pltpu.CompilerParams(
            dimension_semantics=("arbitrary", "arbitrary"),
        ),
    )(a01, bT, p3T)
    return out[0, 0]
